# baseline (speedup 1.0000x reference)
import functools

import jax
import jax.numpy as jnp
from jax import lax
from jax.experimental import pallas as pl
from jax.experimental.pallas import tpu as pltpu
from jax.experimental.pallas import tpu_sc as plsc


def kernel(x, table):
    S, D = table.shape
    info = plsc.get_sparse_core_info()
    NC = info.num_cores
    half = S // NC

    mesh = plsc.ScalarSubcoreMesh(axis_name="c", num_cores=NC)

    @functools.partial(
        pl.kernel,
        mesh=mesh,
        out_type=jax.ShapeDtypeStruct((S, D), jnp.float32),
    )
    def copy_k(table_hbm, out_hbm):
        cid = lax.axis_index("c")
        base = cid * half
        pltpu.sync_copy(table_hbm.at[pl.ds(base, half)],
                        out_hbm.at[pl.ds(base, half)])

    return copy_k(table)[None]


# speedup vs baseline: 6.4424x; 6.4424x over previous
import functools

import jax
import jax.numpy as jnp
from jax import lax
from jax.experimental import pallas as pl
from jax.experimental.pallas import tpu as pltpu
from jax.experimental.pallas import tpu_sc as plsc

NCHUNK = 4


def kernel(x, table):
    S, D = table.shape
    info = plsc.get_sparse_core_info()
    NC = info.num_cores
    half = S // NC
    chunk = half // NCHUNK

    mesh = plsc.ScalarSubcoreMesh(axis_name="c", num_cores=NC)

    @functools.partial(
        pl.kernel,
        mesh=mesh,
        out_type=jax.ShapeDtypeStruct((S, D), jnp.float32),
        scratch_types=[
            pltpu.VMEM_SHARED((NCHUNK, chunk, D), jnp.float32),
            pltpu.SemaphoreType.DMA,
            pltpu.SemaphoreType.DMA,
        ],
    )
    def copy_k(table_hbm, out_hbm, buf, sem_in, sem_out):
        cid = lax.axis_index("c")
        base = cid * half
        gathers = [
            pltpu.async_copy(
                table_hbm.at[pl.ds(base + i * chunk, chunk)], buf.at[i], sem_in)
            for i in range(NCHUNK)
        ]
        scatters = []
        for i in range(NCHUNK):
            gathers[i].wait()
            scatters.append(pltpu.async_copy(
                buf.at[i], out_hbm.at[pl.ds(base + i * chunk, chunk)], sem_out))
        for s in scatters:
            s.wait()

    return copy_k(table)[None]


# speedup vs baseline: 6.4651x; 1.0035x over previous
import functools

import jax
import jax.numpy as jnp
from jax import lax
from jax.experimental import pallas as pl
from jax.experimental.pallas import tpu as pltpu
from jax.experimental.pallas import tpu_sc as plsc

NCHUNK = 2


def kernel(x, table):
    S, D = table.shape
    info = plsc.get_sparse_core_info()
    NC, NS = info.num_cores, info.num_subcores
    NW = NC * NS
    rows_per_w = S // NW
    chunk = rows_per_w // NCHUNK

    mesh = plsc.VectorSubcoreMesh(core_axis_name="c", subcore_axis_name="s")

    @functools.partial(
        pl.kernel,
        mesh=mesh,
        out_type=jax.ShapeDtypeStruct((S, D), jnp.float32),
        scratch_types=[
            pltpu.VMEM((NCHUNK, chunk, D), jnp.float32),
            pltpu.SemaphoreType.DMA,
            pltpu.SemaphoreType.DMA,
        ],
    )
    def copy_k(table_hbm, out_hbm, buf, sem_in, sem_out):
        wid = lax.axis_index("s") * NC + lax.axis_index("c")
        base = wid * rows_per_w
        gathers = [
            pltpu.async_copy(
                table_hbm.at[pl.ds(base + i * chunk, chunk)], buf.at[i], sem_in)
            for i in range(NCHUNK)
        ]
        scatters = []
        for i in range(NCHUNK):
            gathers[i].wait()
            scatters.append(pltpu.async_copy(
                buf.at[i], out_hbm.at[pl.ds(base + i * chunk, chunk)], sem_out))
        for s in scatters:
            s.wait()

    return copy_k(table)[None]
